# baseline (speedup 1.0000x reference)
import functools

import jax
import jax.numpy as jnp
from jax.experimental import pallas as pl
from jax.experimental.pallas import tpu as pltpu

_ROTS = (13, 15, 26, 6, 17, 29, 16, 24)
_KS = (0, 1, 0x1BD11BDB)

_LANES = 128
_NEG_INF = float("-inf")


def _threefry_bits(flat_idx_u32):
    u32 = lambda v: jnp.uint32(v)
    x0 = jnp.zeros_like(flat_idx_u32) + u32(_KS[0])
    x1 = flat_idx_u32 + u32(_KS[1])
    for i in range(5):
        rset = _ROTS[:4] if i % 2 == 0 else _ROTS[4:]
        for r in rset:
            x0 = x0 + x1
            x1 = (x1 << u32(r)) | (x1 >> u32(32 - r))
            x1 = x1 ^ x0
        x0 = x0 + u32(_KS[(i + 1) % 3])
        x1 = x1 + u32(_KS[(i + 2) % 3] + (i + 1))
    return x0 ^ x1


def _gumbel(flat_idx_u32):
    bits = _threefry_bits(flat_idx_u32)
    fbits = (bits >> jnp.uint32(9)) | jnp.uint32(0x3F800000)
    floats = jax.lax.bitcast_convert_type(fbits, jnp.float32) - jnp.float32(1.0)
    minv = jnp.float32(1e-9)
    u = jnp.maximum(minv, floats + minv)
    return -jnp.log(-jnp.log(u))


def _argmax_kernel(logits_ref, idx_ref, vmax_ref, vcol_ref, *, rows, bv,
                   vocab, nblk):
    nb = pl.program_id(1)

    @pl.when(nb == 0)
    def _init():
        vmax_ref[...] = jnp.full((rows, _LANES), _NEG_INF, jnp.float32)
        vcol_ref[...] = jnp.zeros((rows, _LANES), jnp.int32)

    base = nb * bv
    row0 = pl.program_id(0) * rows
    row_i = jax.lax.broadcasted_iota(jnp.int32, (rows, bv), 0) + row0
    col = jax.lax.broadcasted_iota(jnp.int32, (rows, bv), 1) + base
    flat = (row_i * vocab + col).astype(jnp.uint32)

    z = logits_ref[...] + _gumbel(flat)
    z = jnp.where(col < vocab, z, _NEG_INF)

    lane = jax.lax.broadcasted_iota(jnp.int32, (rows, _LANES), 1)
    rv = vmax_ref[...]
    rc = vcol_ref[...]
    for k in range(bv // _LANES):
        cand = z[:, k * _LANES:(k + 1) * _LANES]
        ccol = lane + (base + k * _LANES)
        better = cand > rv
        rv = jnp.where(better, cand, rv)
        rc = jnp.where(better, ccol, rc)
    vmax_ref[...] = rv
    vcol_ref[...] = rc

    @pl.when(nb == nblk - 1)
    def _finish():
        rowmax = jnp.max(rv, axis=1, keepdims=True)
        ccand = jnp.where(rv == rowmax, rc, jnp.int32(2**31 - 1))
        best = jnp.min(ccand, axis=1, keepdims=True)
        idx_ref[...] = jnp.broadcast_to(best, (rows, _LANES))


def _onehot_kernel(idx_ref, out_ref, *, bv):
    nb = pl.program_id(0)
    col = jax.lax.broadcasted_iota(jnp.int32, out_ref.shape, 1) + nb * bv
    idxb = idx_ref[:, 0:1]
    out_ref[...] = jnp.where(col == idxb, jnp.float32(1.0), jnp.float32(0.0))


def kernel(logits):
    batch, vocab = logits.shape
    groups = 2
    rows = batch // groups
    bv = 8192
    nblk = pl.cdiv(vocab, bv)

    idx = pl.pallas_call(
        functools.partial(_argmax_kernel, rows=rows, bv=bv, vocab=vocab,
                          nblk=nblk),
        grid=(groups, nblk),
        in_specs=[pl.BlockSpec((rows, bv), lambda g, nb: (g, nb))],
        out_specs=pl.BlockSpec((rows, _LANES), lambda g, nb: (g, 0)),
        out_shape=jax.ShapeDtypeStruct((batch, _LANES), jnp.int32),
        scratch_shapes=[
            pltpu.VMEM((rows, _LANES), jnp.float32),
            pltpu.VMEM((rows, _LANES), jnp.int32),
        ],
        compiler_params=pltpu.CompilerParams(
            dimension_semantics=("parallel", "arbitrary")),
    )(logits)

    bv2 = 32768
    nblk2 = pl.cdiv(vocab, bv2)
    out = pl.pallas_call(
        functools.partial(_onehot_kernel, bv=bv2),
        grid=(nblk2,),
        in_specs=[pl.BlockSpec((batch, _LANES), lambda nb: (0, 0))],
        out_specs=pl.BlockSpec((batch, bv2), lambda nb: (0, nb)),
        out_shape=jax.ShapeDtypeStruct((batch, vocab), jnp.float32),
        compiler_params=pltpu.CompilerParams(
            dimension_semantics=("parallel",)),
    )(idx)
    return out


# speedup vs baseline: 1.4868x; 1.4868x over previous
import functools

import jax
import jax.numpy as jnp
from jax.experimental import pallas as pl
from jax.experimental.pallas import tpu as pltpu

_ROTS = (13, 15, 26, 6, 17, 29, 16, 24)
_KS = (0, 1, 0x1BD11BDB)

_LANES = 128
_NEG_INF = float("-inf")


def _rotl(x, r):
    return (x << jnp.uint32(r)) | (x >> jnp.uint32(32 - r))


def _threefry_bits(flat_idx_u32):
    u32 = lambda v: jnp.uint32(v)
    x1 = flat_idx_u32 + u32(_KS[1])
    x0 = x1
    x1 = _rotl(x1, _ROTS[0]) ^ x0
    for r in _ROTS[1:4]:
        x0 = x0 + x1
        x1 = _rotl(x1, r) ^ x0
    x0 = x0 + u32(_KS[1])
    x1 = x1 + u32(_KS[2] + 1)
    for i in range(1, 5):
        for r in (_ROTS[4:] if i % 2 == 1 else _ROTS[:4]):
            x0 = x0 + x1
            x1 = _rotl(x1, r) ^ x0
        x0 = x0 + u32(_KS[(i + 1) % 3])
        x1 = x1 + u32(_KS[(i + 2) % 3] + (i + 1))
    return x0 ^ x1


def _gumbel(flat_idx_u32):
    bits = _threefry_bits(flat_idx_u32)
    fbits = (bits >> jnp.uint32(9)) | jnp.uint32(0x3F800000)
    floats = jax.lax.bitcast_convert_type(fbits, jnp.float32) - jnp.float32(1.0)
    minv = jnp.float32(1e-9)
    u = jnp.maximum(minv, floats + minv)
    return -jnp.log(-jnp.log(u))


_SLAB = 512


def _argmax_kernel(logits_ref, idx_ref, vmax_ref, vcol_ref, *, rows, bv,
                   vocab, nblk):
    nb = pl.program_id(1)

    @pl.when(nb == 0)
    def _init():
        vmax_ref[...] = jnp.full((rows, _SLAB), _NEG_INF, jnp.float32)
        vcol_ref[...] = jnp.zeros((rows, _SLAB), jnp.int32)

    base = nb * bv
    row0 = pl.program_id(0) * rows
    row_mul = (jax.lax.broadcasted_iota(jnp.int32, (rows, _SLAB), 0)
               + row0) * vocab
    lane = jax.lax.broadcasted_iota(jnp.int32, (rows, _SLAB), 1)

    rv = vmax_ref[...]
    rc = vcol_ref[...]
    for s in range(bv // _SLAB):
        cols = lane + (base + s * _SLAB)
        flat = (row_mul + cols).astype(jnp.uint32)
        z = logits_ref[:, s * _SLAB:(s + 1) * _SLAB] + _gumbel(flat)
        z = jnp.where(cols < vocab, z, _NEG_INF)
        better = z > rv
        rv = jnp.where(better, z, rv)
        rc = jnp.where(better, cols, rc)
    vmax_ref[...] = rv
    vcol_ref[...] = rc

    @pl.when(nb == nblk - 1)
    def _finish():
        rowmax = jnp.max(rv, axis=1, keepdims=True)
        ccand = jnp.where(rv == rowmax, rc, jnp.int32(2**31 - 1))
        best = jnp.min(ccand, axis=1, keepdims=True)
        idx_ref[...] = jnp.broadcast_to(best, (rows, _LANES))


def _onehot_kernel(idx_ref, out_ref, *, bv):
    nb = pl.program_id(0)
    col = jax.lax.broadcasted_iota(jnp.int32, out_ref.shape, 1) + nb * bv
    idxb = idx_ref[:, 0:1]
    out_ref[...] = jnp.where(col == idxb, jnp.float32(1.0), jnp.float32(0.0))


def kernel(logits):
    batch, vocab = logits.shape
    groups = 2
    rows = batch // groups
    bv = 16384
    nblk = pl.cdiv(vocab, bv)

    idx = pl.pallas_call(
        functools.partial(_argmax_kernel, rows=rows, bv=bv, vocab=vocab,
                          nblk=nblk),
        grid=(groups, nblk),
        in_specs=[pl.BlockSpec((rows, bv), lambda g, nb: (g, nb))],
        out_specs=pl.BlockSpec((rows, _LANES), lambda g, nb: (g, 0)),
        out_shape=jax.ShapeDtypeStruct((batch, _LANES), jnp.int32),
        scratch_shapes=[
            pltpu.VMEM((rows, _SLAB), jnp.float32),
            pltpu.VMEM((rows, _SLAB), jnp.int32),
        ],
        compiler_params=pltpu.CompilerParams(
            dimension_semantics=("parallel", "arbitrary")),
    )(logits)

    bv2 = 32768
    nblk2 = pl.cdiv(vocab, bv2)
    out = pl.pallas_call(
        functools.partial(_onehot_kernel, bv=bv2),
        grid=(nblk2,),
        in_specs=[pl.BlockSpec((batch, _LANES), lambda nb: (0, 0))],
        out_specs=pl.BlockSpec((batch, bv2), lambda nb: (0, nb)),
        out_shape=jax.ShapeDtypeStruct((batch, vocab), jnp.float32),
        compiler_params=pltpu.CompilerParams(
            dimension_semantics=("parallel",)),
    )(idx)
    return out
